# baseline (speedup 1.0000x reference)
import jax
import jax.numpy as jnp
from jax.experimental import pallas as pl

B, N, D = 32, 128, 128
SIGMA = 2.5


def _body(ct_o, c_o, x_o, ct_m, c_m, x_m,
          w1, b1, w2, b2, wt1, bt1, wt2, bt2, out):
    def pooled(ct_ref, c_ref, x_ref):
        ct = ct_ref[0]
        c = c_ref[0]
        x = x_ref[0]

        d2 = (c[:, 0:1] - ct[0:1, :]) ** 2
        d2 += (c[:, 1:2] - ct[1:2, :]) ** 2
        d2 += (c[:, 2:3] - ct[2:3, :]) ** 2
        dist = jnp.sqrt(d2 + 1e-12)
        A = jnp.exp(dist * (-1.0 / SIGMA))
        ii = jax.lax.broadcasted_iota(jnp.int32, (N, N), 0)
        jj = jax.lax.broadcasted_iota(jnp.int32, (N, N), 1)
        A = jnp.where(ii == jj, jnp.float32(1.0), A)

        deg = jnp.sum(A, axis=1, keepdims=True)
        dinv = 1.0 / jnp.sqrt(deg)

        h = x
        for w_ref, b_ref in ((w1, b1), (w2, b2)):
            y = h * dinv
            z = jnp.dot(A, y, preferred_element_type=jnp.float32)
            z = z * dinv
            z = jnp.dot(z, w_ref[...], preferred_element_type=jnp.float32)
            h = jnp.maximum(z + b_ref[...], 0.0)
        return jnp.max(h, axis=0, keepdims=True)

    po = pooled(ct_o, c_o, x_o)
    pm = pooled(ct_m, c_m, x_m)

    t = jnp.dot(po, wt1[0:D, :], preferred_element_type=jnp.float32)
    t += jnp.dot(pm, wt1[D:2 * D, :], preferred_element_type=jnp.float32)
    t = jnp.maximum(t + bt1[...], 0.0)
    s = jnp.sum(t * wt2[...]) + bt2[0, 0]
    out[...] = jnp.zeros((1, 1, D), jnp.float32) + s


def kernel(coords_orig, feats_orig, coords_mut, feats_mut,
           W1, b1, W2, b2, Wt1, bt1, Wt2, bt2):
    ct_o = jnp.swapaxes(coords_orig, 1, 2)
    ct_m = jnp.swapaxes(coords_mut, 1, 2)

    per_graph = lambda i: (i, 0, 0)
    const2 = lambda i: (0, 0)

    return pl.pallas_call(
        _body,
        grid=(B,),
        in_specs=[
            pl.BlockSpec((1, 3, N), per_graph),
            pl.BlockSpec((1, N, 3), per_graph),
            pl.BlockSpec((1, N, D), per_graph),
            pl.BlockSpec((1, 3, N), per_graph),
            pl.BlockSpec((1, N, 3), per_graph),
            pl.BlockSpec((1, N, D), per_graph),
            pl.BlockSpec((D, D), const2),
            pl.BlockSpec((1, D), const2),
            pl.BlockSpec((D, D), const2),
            pl.BlockSpec((1, D), const2),
            pl.BlockSpec((2 * D, D), const2),
            pl.BlockSpec((1, D), const2),
            pl.BlockSpec((1, D), const2),
            pl.BlockSpec((1, 1), const2),
        ],
        out_specs=pl.BlockSpec((1, 1, D), lambda i: (i, 0, 0)),
        out_shape=jax.ShapeDtypeStruct((B, 1, D), jnp.float32),
    )(ct_o, coords_orig, feats_orig, ct_m, coords_mut, feats_mut,
      W1, b1.reshape(1, D), W2, b2.reshape(1, D),
      Wt1, bt1.reshape(1, D), Wt2.reshape(1, D), bt2.reshape(1, 1))[:, 0, :1]


# speedup vs baseline: 1.6586x; 1.6586x over previous
import jax
import jax.numpy as jnp
from jax.experimental import pallas as pl

B, N, D = 32, 128, 128
G = 8
SIGMA = 2.5


def _body(ct_o, c_o, x_o, ct_m, c_m, x_m,
          w1, b1, w2, b2, wt1, bt1, wt2, bt2, out):
    w1v = w1[...]
    w2v = w2[...]
    b1v = b1[...]
    b2v = b2[...]

    def pooled(ct, c, x):
        d2 = (c[:, 0:1] - ct[0:1, :]) ** 2
        d2 += (c[:, 1:2] - ct[1:2, :]) ** 2
        d2 += (c[:, 2:3] - ct[2:3, :]) ** 2
        dist = jnp.sqrt(d2 + 1e-12)
        A = jnp.exp(dist * (-1.0 / SIGMA))
        ii = jax.lax.broadcasted_iota(jnp.int32, (N, N), 0)
        jj = jax.lax.broadcasted_iota(jnp.int32, (N, N), 1)
        A = jnp.where(ii == jj, jnp.float32(1.0), A)

        deg = jnp.sum(A, axis=1, keepdims=True)
        dinv = 1.0 / jnp.sqrt(deg)

        h = x
        for wv, bv in ((w1v, b1v), (w2v, b2v)):
            y = h * dinv
            z = jnp.dot(A, y, preferred_element_type=jnp.float32)
            z = z * dinv
            z = jnp.dot(z, wv, preferred_element_type=jnp.float32)
            h = jnp.maximum(z + bv, 0.0)
        return jnp.max(h, axis=0, keepdims=True)

    po = jnp.concatenate(
        [pooled(ct_o[g], c_o[g], x_o[g]) for g in range(G)], axis=0)
    pm = jnp.concatenate(
        [pooled(ct_m[g], c_m[g], x_m[g]) for g in range(G)], axis=0)

    t = jnp.dot(po, wt1[0:D, :], preferred_element_type=jnp.float32)
    t += jnp.dot(pm, wt1[D:2 * D, :], preferred_element_type=jnp.float32)
    t = jnp.maximum(t + bt1[...], 0.0)
    s = jnp.sum(t * wt2[...], axis=1, keepdims=True) + bt2[0, 0]
    out[...] = jnp.zeros((G, D), jnp.float32) + s


def kernel(coords_orig, feats_orig, coords_mut, feats_mut,
           W1, b1, W2, b2, Wt1, bt1, Wt2, bt2):
    ct_o = jnp.swapaxes(coords_orig, 1, 2)
    ct_m = jnp.swapaxes(coords_mut, 1, 2)

    per_chunk = lambda i: (i, 0, 0)
    const2 = lambda i: (0, 0)

    return pl.pallas_call(
        _body,
        grid=(B // G,),
        in_specs=[
            pl.BlockSpec((G, 3, N), per_chunk),
            pl.BlockSpec((G, N, 3), per_chunk),
            pl.BlockSpec((G, N, D), per_chunk),
            pl.BlockSpec((G, 3, N), per_chunk),
            pl.BlockSpec((G, N, 3), per_chunk),
            pl.BlockSpec((G, N, D), per_chunk),
            pl.BlockSpec((D, D), const2),
            pl.BlockSpec((1, D), const2),
            pl.BlockSpec((D, D), const2),
            pl.BlockSpec((1, D), const2),
            pl.BlockSpec((2 * D, D), const2),
            pl.BlockSpec((1, D), const2),
            pl.BlockSpec((1, D), const2),
            pl.BlockSpec((1, 1), const2),
        ],
        out_specs=pl.BlockSpec((G, D), lambda i: (i, 0)),
        out_shape=jax.ShapeDtypeStruct((B, D), jnp.float32),
    )(ct_o, coords_orig, feats_orig, ct_m, coords_mut, feats_mut,
      W1, b1.reshape(1, D), W2, b2.reshape(1, D),
      Wt1, bt1.reshape(1, D), Wt2.reshape(1, D), bt2.reshape(1, 1))[:, :1]
